# Initial kernel scaffold; baseline (speedup 1.0000x reference)
import functools

import jax
import jax.numpy as jnp
from jax import lax
from jax.experimental import pallas as pl
from jax.experimental.pallas import tpu as pltpu
from jax.experimental.pallas import tpu_sc as plsc

_DIM = 128
_BATCH = 16384
_MARGIN = 1.0
_NC = 2
_NS = 16
_NW = _NC * _NS
_CHUNK = 128
_PER_W = _BATCH // _NW
_NCHUNK = _PER_W // _CHUNK
_L = 16
_NJ = _DIM // _L


def _rsqrt(x):
    i = lax.bitcast_convert_type(x, jnp.int32)
    i = jnp.int32(0x5F3759DF) - lax.shift_right_arithmetic(i, 1)
    y = lax.bitcast_convert_type(i, jnp.float32)
    for _ in range(3):
        y = y * (jnp.float32(1.5) - jnp.float32(0.5) * x * y * y)
    return y


def _sqrt(x):
    s = jnp.maximum(x, jnp.float32(1e-30))
    return s * _rsqrt(s)


def _sc_body(ent_hbm, rel_hbm, hi_hbm, ri_hbm, ti_hbm, hci_hbm, tci_hbm,
             out_hbm,
             hi_v, ri_v, ti_v, hci_v, tci_v,
             h_v, r_v, t_v, hc_v, tc_v, acc_v, sem):
    wid = lax.axis_index("s") * _NC + lax.axis_index("c")
    pltpu.sync_copy(hi_hbm.at[wid], hi_v)
    pltpu.sync_copy(ri_hbm.at[wid], ri_v)
    pltpu.sync_copy(ti_hbm.at[wid], ti_v)
    pltpu.sync_copy(hci_hbm.at[wid], hci_v)
    pltpu.sync_copy(tci_hbm.at[wid], tci_v)

    def chunk_body(c, acc):
        cps = [
            pltpu.async_copy(ent_hbm.at[hi_v.at[c]], h_v, sem),
            pltpu.async_copy(rel_hbm.at[ri_v.at[c]], r_v, sem),
            pltpu.async_copy(ent_hbm.at[ti_v.at[c]], t_v, sem),
            pltpu.async_copy(ent_hbm.at[hci_v.at[c]], hc_v, sem),
            pltpu.async_copy(ent_hbm.at[tci_v.at[c]], tc_v, sem),
        ]
        for cp in cps:
            cp.wait()

        def elem_body(e, a):
            z = jnp.zeros((_L,), jnp.float32)
            d_hh = d_tt = d_rr = d_hr = d_ht = d_rt = z
            d_cc = d_uu = d_cr = d_cu = d_ru = z
            for j in range(_NJ):
                sl = pl.ds(j * _L, _L)
                h = h_v[e, sl]
                r = r_v[e, sl]
                t = t_v[e, sl]
                hc = hc_v[e, sl]
                tc = tc_v[e, sl]
                d_hh = d_hh + h * h
                d_tt = d_tt + t * t
                d_rr = d_rr + r * r
                d_hr = d_hr + h * r
                d_ht = d_ht + h * t
                d_rt = d_rt + r * t
                d_cc = d_cc + hc * hc
                d_uu = d_uu + tc * tc
                d_cr = d_cr + hc * r
                d_cu = d_cu + hc * tc
                d_ru = d_ru + r * tc
            sh = jnp.sum(d_hh)
            st = jnp.sum(d_tt)
            rr = jnp.sum(d_rr)
            hr = jnp.sum(d_hr)
            ht = jnp.sum(d_ht)
            rt = jnp.sum(d_rt)
            shc = jnp.sum(d_cc)
            stc = jnp.sum(d_uu)
            hcr = jnp.sum(d_cr)
            hctc = jnp.sum(d_cu)
            rtc = jnp.sum(d_ru)
            ih = _rsqrt(sh)
            it = _rsqrt(st)
            ihc = _rsqrt(shc)
            itc = _rsqrt(stc)
            two = jnp.float32(2.0)
            spos = two + rr + two * (hr * ih - ht * (ih * it) - rt * it)
            sneg = two + rr + two * (hcr * ihc - hctc * (ihc * itc) - rtc * itc)
            contrib = jnp.maximum(jnp.float32(0.0),
                                  jnp.float32(_MARGIN) + _sqrt(spos) - _sqrt(sneg))
            return a + contrib

        return lax.fori_loop(0, _CHUNK, elem_body, acc)

    loss = lax.fori_loop(0, _NCHUNK, chunk_body, jnp.float32(0.0))
    acc_v[...] = jnp.broadcast_to(loss * jnp.float32(1.0 / _BATCH), (_L,))
    pltpu.sync_copy(acc_v, out_hbm.at[wid])


_sc_fn = functools.partial(
    pl.kernel,
    out_type=jax.ShapeDtypeStruct((_NW, _L), jnp.float32),
    mesh=plsc.VectorSubcoreMesh(core_axis_name="c", subcore_axis_name="s"),
    scratch_types=[
        pltpu.VMEM((_NCHUNK, _CHUNK), jnp.int32),
        pltpu.VMEM((_NCHUNK, _CHUNK), jnp.int32),
        pltpu.VMEM((_NCHUNK, _CHUNK), jnp.int32),
        pltpu.VMEM((_NCHUNK, _CHUNK), jnp.int32),
        pltpu.VMEM((_NCHUNK, _CHUNK), jnp.int32),
        pltpu.VMEM((_CHUNK, _DIM), jnp.float32),
        pltpu.VMEM((_CHUNK, _DIM), jnp.float32),
        pltpu.VMEM((_CHUNK, _DIM), jnp.float32),
        pltpu.VMEM((_CHUNK, _DIM), jnp.float32),
        pltpu.VMEM((_CHUNK, _DIM), jnp.float32),
        pltpu.VMEM((_L,), jnp.float32),
        pltpu.SemaphoreType.DMA,
    ],
)(_sc_body)


def kernel(entity_emb, relation_emb, triplets, corrupted_triplets):
    shp = (_NW, _NCHUNK, _CHUNK)
    h_i = triplets[:, 0].reshape(shp)
    r_i = triplets[:, 1].reshape(shp)
    t_i = triplets[:, 2].reshape(shp)
    hc_i = corrupted_triplets[:, 0].reshape(shp)
    tc_i = corrupted_triplets[:, 2].reshape(shp)
    out = _sc_fn(entity_emb, relation_emb, h_i, r_i, t_i, hc_i, tc_i)
    return jnp.sum(out[:, 0])


# speedup vs baseline: 3.5338x; 3.5338x over previous
import functools

import jax
import jax.numpy as jnp
from jax import lax
from jax.experimental import pallas as pl
from jax.experimental.pallas import tpu as pltpu
from jax.experimental.pallas import tpu_sc as plsc

_DIM = 128
_BATCH = 16384
_MARGIN = 1.0
_NC = 2
_NS = 16
_NW = _NC * _NS
_CHUNK = 128
_PER_W = _BATCH // _NW
_NCHUNK = _PER_W // _CHUNK
_L = 16
_NJ = _DIM // _L


def _rsqrt(x):
    i = lax.bitcast_convert_type(x, jnp.int32)
    i = jnp.int32(0x5F3759DF) - lax.shift_right_arithmetic(i, 1)
    y = lax.bitcast_convert_type(i, jnp.float32)
    for _ in range(3):
        y = y * (jnp.float32(1.5) - jnp.float32(0.5) * x * y * y)
    return y


def _sqrt(x):
    s = jnp.maximum(x, jnp.float32(1e-30))
    return s * _rsqrt(s)


def _sc_body(ent_hbm, rel_hbm, hi_hbm, ri_hbm, ti_hbm, hci_hbm, tci_hbm,
             out_hbm,
             hi_v, ri_v, ti_v, hci_v, tci_v,
             h_v, r_v, t_v, hc_v, tc_v, acc_v, sem):
    wid = lax.axis_index("s") * _NC + lax.axis_index("c")
    pltpu.sync_copy(hi_hbm.at[wid], hi_v)
    pltpu.sync_copy(ri_hbm.at[wid], ri_v)
    pltpu.sync_copy(ti_hbm.at[wid], ti_v)
    pltpu.sync_copy(hci_hbm.at[wid], hci_v)
    pltpu.sync_copy(tci_hbm.at[wid], tci_v)

    def chunk_body(c, acc):
        cps = [
            pltpu.async_copy(ent_hbm.at[hi_v.at[c]], h_v, sem),
            pltpu.async_copy(rel_hbm.at[ri_v.at[c]], r_v, sem),
            pltpu.async_copy(ent_hbm.at[ti_v.at[c]], t_v, sem),
            pltpu.async_copy(ent_hbm.at[hci_v.at[c]], hc_v, sem),
            pltpu.async_copy(ent_hbm.at[tci_v.at[c]], tc_v, sem),
        ]
        for cp in cps:
            cp.wait()

        def elem_body(e, a):
            z = jnp.zeros((_L,), jnp.float32)
            d_hh = d_tt = d_rr = d_hr = d_ht = d_rt = z
            d_cc = d_uu = d_cr = d_cu = d_ru = z
            for j in range(_NJ):
                sl = pl.ds(j * _L, _L)
                h = h_v[e, sl]
                r = r_v[e, sl]
                t = t_v[e, sl]
                hc = hc_v[e, sl]
                tc = tc_v[e, sl]
                d_hh = d_hh + h * h
                d_tt = d_tt + t * t
                d_rr = d_rr + r * r
                d_hr = d_hr + h * r
                d_ht = d_ht + h * t
                d_rt = d_rt + r * t
                d_cc = d_cc + hc * hc
                d_uu = d_uu + tc * tc
                d_cr = d_cr + hc * r
                d_cu = d_cu + hc * tc
                d_ru = d_ru + r * tc
            sh = jnp.sum(d_hh)
            st = jnp.sum(d_tt)
            rr = jnp.sum(d_rr)
            hr = jnp.sum(d_hr)
            ht = jnp.sum(d_ht)
            rt = jnp.sum(d_rt)
            shc = jnp.sum(d_cc)
            stc = jnp.sum(d_uu)
            hcr = jnp.sum(d_cr)
            hctc = jnp.sum(d_cu)
            rtc = jnp.sum(d_ru)
            ih = _rsqrt(sh)
            it = _rsqrt(st)
            ihc = _rsqrt(shc)
            itc = _rsqrt(stc)
            two = jnp.float32(2.0)
            spos = two + rr + two * (hr * ih - ht * (ih * it) - rt * it)
            sneg = two + rr + two * (hcr * ihc - hctc * (ihc * itc) - rtc * itc)
            contrib = jnp.maximum(jnp.float32(0.0),
                                  jnp.float32(_MARGIN) + _sqrt(spos) - _sqrt(sneg))
            return a + contrib

        return lax.fori_loop(0, _CHUNK, elem_body, acc)

    loss = lax.fori_loop(0, _NCHUNK, chunk_body, jnp.float32(0.0))
    acc_v[...] = jnp.broadcast_to(loss * jnp.float32(1.0 / _BATCH), (_L,))
    pltpu.sync_copy(acc_v, out_hbm.at[wid])


_sc_fn = functools.partial(
    pl.kernel,
    out_type=jax.ShapeDtypeStruct((_NW, _L), jnp.float32),
    mesh=plsc.VectorSubcoreMesh(core_axis_name="c", subcore_axis_name="s"),
    compiler_params=pltpu.CompilerParams(needs_layout_passes=False),
    scratch_types=[
        pltpu.VMEM((_NCHUNK, _CHUNK), jnp.int32),
        pltpu.VMEM((_NCHUNK, _CHUNK), jnp.int32),
        pltpu.VMEM((_NCHUNK, _CHUNK), jnp.int32),
        pltpu.VMEM((_NCHUNK, _CHUNK), jnp.int32),
        pltpu.VMEM((_NCHUNK, _CHUNK), jnp.int32),
        pltpu.VMEM((_CHUNK, _DIM), jnp.float32),
        pltpu.VMEM((_CHUNK, _DIM), jnp.float32),
        pltpu.VMEM((_CHUNK, _DIM), jnp.float32),
        pltpu.VMEM((_CHUNK, _DIM), jnp.float32),
        pltpu.VMEM((_CHUNK, _DIM), jnp.float32),
        pltpu.VMEM((_L,), jnp.float32),
        pltpu.SemaphoreType.DMA,
    ],
)(_sc_body)


def kernel(entity_emb, relation_emb, triplets, corrupted_triplets):
    shp = (_NW, _NCHUNK, _CHUNK)
    h_i = triplets[:, 0].reshape(shp)
    r_i = triplets[:, 1].reshape(shp)
    t_i = triplets[:, 2].reshape(shp)
    hc_i = corrupted_triplets[:, 0].reshape(shp)
    tc_i = corrupted_triplets[:, 2].reshape(shp)
    out = _sc_fn(entity_emb, relation_emb, h_i, r_i, t_i, hc_i, tc_i)
    return jnp.sum(out[:, 0])
